# baseline (speedup 1.0000x reference)
import functools

import jax
import jax.numpy as jnp
from jax import lax
from jax.experimental import pallas as pl
from jax.experimental.pallas import tpu as pltpu
from jax.experimental.pallas import tpu_sc as plsc

BATCH = 16384
TIMESTEPS = 200
EMBED_DIM = 32
N = BATCH * TIMESTEPS
NUM_CORES = 2
NUM_SUBCORES = 16
NUM_WORKERS = NUM_CORES * NUM_SUBCORES
PER_WORKER = N // NUM_WORKERS
NBUF = 4
K = NBUF - 2
CHUNK = 800
NCHUNK = PER_WORKER // CHUNK
NOUT = NCHUNK // NBUF

_mesh = plsc.VectorSubcoreMesh(core_axis_name="c", subcore_axis_name="s")


@functools.partial(
    pl.kernel,
    mesh=_mesh,
    out_type=jax.ShapeDtypeStruct((N, EMBED_DIM), jnp.float32),
    scratch_types=[
        pltpu.VMEM((NBUF, CHUNK), jnp.int32),
        pltpu.VMEM((NBUF, CHUNK, EMBED_DIM), jnp.float32),
        [pltpu.SemaphoreType.DMA] * NBUF,
        [pltpu.SemaphoreType.DMA] * NBUF,
        [pltpu.SemaphoreType.DMA] * NBUF,
    ],
    compiler_params=pltpu.CompilerParams(use_tc_tiling_on_sc=False),
)
def _emb_lookup(items_hbm, table_hbm, out_hbm, idx_v, rows_v,
                idx_sems, gat_sems, out_sems):
    wid = lax.axis_index("s") * NUM_CORES + lax.axis_index("c")
    base = wid * PER_WORKER

    def start_idx(c, b):
        off = base + c * CHUNK
        pltpu.async_copy(items_hbm.at[pl.ds(off, CHUNK)], idx_v.at[b],
                         idx_sems[b])

    def wait_idx(b):
        pltpu.make_async_copy(items_hbm.at[pl.ds(base, CHUNK)], idx_v.at[b],
                              idx_sems[b]).wait()

    def start_gather(b):
        pltpu.async_copy(table_hbm.at[idx_v.at[b]], rows_v.at[b], gat_sems[b])

    def wait_gather(b):
        pltpu.make_async_copy(table_hbm.at[idx_v.at[b]], rows_v.at[b],
                              gat_sems[b]).wait()

    def start_store(c, b):
        off = base + c * CHUNK
        pltpu.async_copy(rows_v.at[b], out_hbm.at[pl.ds(off, CHUNK)],
                         out_sems[b])

    def wait_store(b):
        pltpu.make_async_copy(rows_v.at[b], out_hbm.at[pl.ds(base, CHUNK)],
                              out_sems[b]).wait()

    for b in range(NBUF):
        start_idx(b, b)
    for b in range(NBUF):
        c = b
        wait_idx(b)
        start_gather(b)
        d = c - K
        if d >= 0:
            bd = (b - K) % NBUF
            wait_gather(bd)
            start_store(d, bd)
            start_idx(d + NBUF, bd)

    def body(o, carry):
        for b in range(NBUF):
            c = o * NBUF + b
            wait_store(b)
            wait_idx(b)
            start_gather(b)
            d = c - K
            bd = (b - K) % NBUF
            wait_gather(bd)
            start_store(d, bd)
            start_idx(d + NBUF, bd)
        return carry

    lax.fori_loop(1, NOUT - 1, body, 0)

    for b in range(NBUF):
        c = (NOUT - 1) * NBUF + b
        wait_store(b)
        wait_idx(b)
        start_gather(b)
        d = c - K
        bd = (b - K) % NBUF
        wait_gather(bd)
        start_store(d, bd)
        if d + NBUF < NCHUNK:
            start_idx(d + NBUF, bd)

    for j in range(K):
        d = NCHUNK - K + j
        bd = d % NBUF
        wait_gather(bd)
        start_store(d, bd)
    for b in range(NBUF):
        wait_store(b)


def kernel(items, table):
    flat = items.reshape(N).astype(jnp.int32)
    out = _emb_lookup(flat, table)
    return out.reshape(BATCH, TIMESTEPS, EMBED_DIM)
